# baseline (speedup 1.0000x reference)
import functools

import jax
import jax.numpy as jnp
from jax import lax
from jax.experimental import pallas as pl
from jax.experimental.pallas import tpu as pltpu
from jax.experimental.pallas import tpu_sc as plsc

VOCAB = 100000
HID = 768
MAXPOS = 2048
TYPES = 2
B = 4
S = 2048
C = 4
EPS = 1e-12

N = B * S
NC = 2
NSUB = 16
NW = NC * NSUB
TPW = N // NW
CHUNK = 16
NSTEP = TPW // CHUNK
LANES = 16
HB = HID // LANES

SBLK = 2048
NSB = S // SBLK


def _sc_pool(ids_flat, word_table):
    mesh = plsc.VectorSubcoreMesh(core_axis_name="c", subcore_axis_name="s")

    @functools.partial(
        pl.kernel,
        mesh=mesh,
        compiler_params=pltpu.CompilerParams(needs_layout_passes=False),
        out_type=jax.ShapeDtypeStruct((N, HID), jnp.float32),
        scratch_types=[
            pltpu.VMEM((TPW * C,), jnp.int32),
            pltpu.VMEM((CHUNK * C,), jnp.float32),
            pltpu.VMEM((2, CHUNK * C, HID), jnp.float32),
            pltpu.VMEM((2, CHUNK, HID), jnp.float32),
            pltpu.SemaphoreType.DMA,
            pltpu.SemaphoreType.DMA,
            pltpu.SemaphoreType.DMA,
            pltpu.SemaphoreType.DMA,
        ],
    )
    def k(ids_hbm, word_hbm, out_hbm, idx_all, w_v, rows_v, out_v,
          sem0, sem1, semo0, semo1):
        wid = lax.axis_index("s") * NC + lax.axis_index("c")
        base = wid * TPW
        sems = (sem0, sem1)
        semos = (semo0, semo1)
        GN = CHUNK * C // LANES

        pltpu.sync_copy(ids_hbm.at[pl.ds(base * C, TPW * C)], idx_all)

        def _scan_min(g, m):
            return jnp.minimum(m, idx_all[pl.ds(g * LANES, LANES)])

        mn_all = lax.fori_loop(0, TPW * C // LANES, _scan_min,
                               jnp.full((LANES,), VOCAB, jnp.int32))
        allpos_w = jnp.min(mn_all) >= 1

        def start(si, b):
            pltpu.async_copy(
                word_hbm.at[idx_all.at[pl.ds(si * CHUNK * C, CHUNK * C)]],
                rows_v.at[b], sems[b])

        def wait(b):
            pltpu.make_async_copy(
                word_hbm.at[idx_all.at[pl.ds(0, CHUNK * C)]],
                rows_v.at[b], sems[b]).wait()

        def compute(si, b):
            tok0 = base + si * CHUNK
            off = si * (CHUNK * C)

            @pl.when(si >= 2)
            def _():
                pltpu.make_async_copy(
                    out_v.at[b], out_hbm.at[pl.ds(tok0 - 2 * CHUNK, CHUNK)],
                    semos[b]).wait()

            @pl.when(allpos_w)
            def _fast():
                @plsc.parallel_loop(0, HID, step=LANES)
                def hblk(hoff):
                    sl = pl.ds(hoff, LANES)
                    for t in range(CHUNK):
                        r = t * C
                        v = ((rows_v[b, r, sl] + rows_v[b, r + 1, sl])
                             + (rows_v[b, r + 2, sl] + rows_v[b, r + 3, sl]))
                        out_v[b, t, sl] = v * 0.25

            @pl.when(jnp.logical_not(allpos_w))
            def _slow():
                for g in range(GN):
                    ivg = idx_all[pl.ds(off + g * LANES, LANES)]
                    w_v[pl.ds(g * LANES, LANES)] = (
                        jnp.minimum(ivg, 1).astype(jnp.float32))
                lane = lax.iota(jnp.int32, LANES)
                gidx = [lane * C + c for c in range(C)]
                gw = [plsc.load_gather(w_v, [gidx[c]]) for c in range(C)]
                den = gw[0] + gw[1] + gw[2] + gw[3]
                invd = 1.0 / jnp.maximum(den, 1.0)
                for c in range(C):
                    plsc.store_scatter(w_v, [gidx[c]], gw[c] * invd)

                def tok(t, c0):
                    r = t * C
                    a = [plsc.load_gather(
                        w_v, [jnp.full((LANES,), r + c, jnp.int32)])
                        for c in range(C)]

                    def hblk(h, c1):
                        sl = pl.ds(h * LANES, LANES)
                        v = (rows_v[b, r, sl] * a[0] + rows_v[b, r + 1, sl] * a[1]
                             + rows_v[b, r + 2, sl] * a[2]
                             + rows_v[b, r + 3, sl] * a[3])
                        out_v[b, t, sl] = v
                        return c1

                    return lax.fori_loop(0, HB, hblk, c0)

                lax.fori_loop(0, CHUNK, tok, 0)

            pltpu.async_copy(out_v.at[b], out_hbm.at[pl.ds(tok0, CHUNK)],
                             semos[b])

        start(0, 0)

        def body(i, carry):
            s0 = 2 * i
            start(s0 + 1, 1)
            wait(0)
            compute(s0, 0)

            @pl.when(s0 + 2 < NSTEP)
            def _():
                start(s0 + 2, 0)

            wait(1)
            compute(s0 + 1, 1)
            return carry

        lax.fori_loop(0, NSTEP // 2, body, 0)

        for b in (0, 1):
            pltpu.make_async_copy(
                out_v.at[b],
                out_hbm.at[pl.ds(base + (NSTEP - 2 + b) * CHUNK, CHUNK)],
                semos[b]).wait()

    return k(ids_flat, word_table)


def _tc_ln(pooled, ttf3, pos_table, type_table, ln_gamma, ln_beta):
    def body(pooled_ref, ttf_ref, pos_ref, ty_ref, gam_ref, bet_ref, out_ref):
        x = pooled_ref[...]
        ptt = ttf_ref[0, 0]
        dty = ty_ref[1] - ty_ref[0]
        ty = ty_ref[0][None, :] + ptt[:, None] * dty[None, :]
        e = x + pos_ref[...] + ty
        u = jnp.mean(e, axis=-1, keepdims=True)
        s = jnp.mean((e - u) ** 2, axis=-1, keepdims=True)
        xn = (e - u) * lax.rsqrt(s + EPS)
        out_ref[...] = gam_ref[...][None, :] * xn + bet_ref[...][None, :]

    grid = (NSB, B)
    return pl.pallas_call(
        body,
        grid=grid,
        in_specs=[
            pl.BlockSpec((SBLK, HID), lambda si, bi: (bi * NSB + si, 0)),
            pl.BlockSpec((1, 1, SBLK), lambda si, bi: (bi * NSB + si, 0, 0)),
            pl.BlockSpec((SBLK, HID), lambda si, bi: (si, 0)),
            pl.BlockSpec((TYPES, HID), lambda si, bi: (0, 0)),
            pl.BlockSpec((HID,), lambda si, bi: (0,)),
            pl.BlockSpec((HID,), lambda si, bi: (0,)),
        ],
        out_specs=pl.BlockSpec((SBLK, HID), lambda si, bi: (bi * NSB + si, 0)),
        out_shape=jax.ShapeDtypeStruct((N, HID), jnp.float32),
    )(pooled, ttf3, pos_table, type_table, ln_gamma, ln_beta)


def kernel(input_ids, token_type_ids, token_ids, word_table, pos_table,
           type_table, ln_gamma, ln_beta):
    ids_flat = token_ids.astype(jnp.int32).reshape(N * C)
    pooled = _sc_pool(ids_flat, word_table)
    ttf3 = token_type_ids.astype(jnp.float32).reshape(B * NSB, 1, SBLK)
    out = _tc_ln(pooled, ttf3, pos_table, type_table, ln_gamma, ln_beta)
    return out.reshape(B, S, HID)


# speedup vs baseline: 1.0031x; 1.0031x over previous
import functools

import jax
import jax.numpy as jnp
from jax import lax
from jax.experimental import pallas as pl
from jax.experimental.pallas import tpu as pltpu
from jax.experimental.pallas import tpu_sc as plsc

VOCAB = 100000
HID = 768
MAXPOS = 2048
TYPES = 2
B = 4
S = 2048
C = 4
EPS = 1e-12

N = B * S
NC = 2
NSUB = 16
NW = NC * NSUB
TPW = N // NW
CHUNK = 16
NSTEP = TPW // CHUNK
LANES = 16
HB = HID // LANES

SBLK = 2048
NSB = S // SBLK


def _sc_pool(ids_flat, word_table):
    mesh = plsc.VectorSubcoreMesh(core_axis_name="c", subcore_axis_name="s")

    @functools.partial(
        pl.kernel,
        mesh=mesh,
        compiler_params=pltpu.CompilerParams(needs_layout_passes=False),
        out_type=jax.ShapeDtypeStruct((N, HID), jnp.float32),
        scratch_types=[
            pltpu.VMEM((TPW * C,), jnp.int32),
            pltpu.VMEM((CHUNK * C,), jnp.float32),
            pltpu.VMEM((2, CHUNK * C, HID), jnp.float32),
            pltpu.VMEM((2, CHUNK, HID), jnp.float32),
            pltpu.SemaphoreType.DMA,
            pltpu.SemaphoreType.DMA,
            pltpu.SemaphoreType.DMA,
            pltpu.SemaphoreType.DMA,
        ],
    )
    def k(ids_hbm, word_hbm, out_hbm, idx_all, w_v, rows_v, out_v,
          sem0, sem1, semo0, semo1):
        wid = lax.axis_index("s") * NC + lax.axis_index("c")
        base = wid * TPW
        sems = (sem0, sem1)
        semos = (semo0, semo1)
        GN = CHUNK * C // LANES

        pltpu.sync_copy(ids_hbm.at[pl.ds(base * C, TPW * C)], idx_all)

        def start(si, b):
            pltpu.async_copy(
                word_hbm.at[idx_all.at[pl.ds(si * CHUNK * C, CHUNK * C)]],
                rows_v.at[b], sems[b])

        def wait(b):
            pltpu.make_async_copy(
                word_hbm.at[idx_all.at[pl.ds(0, CHUNK * C)]],
                rows_v.at[b], sems[b]).wait()

        def compute(si, b):
            tok0 = base + si * CHUNK
            off = si * (CHUNK * C)
            iv = [idx_all[pl.ds(off + g * LANES, LANES)] for g in range(GN)]

            @pl.when(si >= 2)
            def _():
                pltpu.make_async_copy(
                    out_v.at[b], out_hbm.at[pl.ds(tok0 - 2 * CHUNK, CHUNK)],
                    semos[b]).wait()

            mn = iv[0]
            for g in range(1, GN):
                mn = jnp.minimum(mn, iv[g])
            allpos = jnp.min(mn) >= 1

            @pl.when(allpos)
            def _fast():
                @plsc.parallel_loop(0, HID, step=LANES)
                def hblk(hoff):
                    sl = pl.ds(hoff, LANES)
                    for t in range(CHUNK):
                        r = t * C
                        v = ((rows_v[b, r, sl] + rows_v[b, r + 1, sl])
                             + (rows_v[b, r + 2, sl] + rows_v[b, r + 3, sl]))
                        out_v[b, t, sl] = v * 0.25

            @pl.when(jnp.logical_not(allpos))
            def _slow():
                for g in range(GN):
                    w_v[pl.ds(g * LANES, LANES)] = (
                        jnp.minimum(iv[g], 1).astype(jnp.float32))
                lane = lax.iota(jnp.int32, LANES)
                gidx = [lane * C + c for c in range(C)]
                gw = [plsc.load_gather(w_v, [gidx[c]]) for c in range(C)]
                den = gw[0] + gw[1] + gw[2] + gw[3]
                invd = 1.0 / jnp.maximum(den, 1.0)
                for c in range(C):
                    plsc.store_scatter(w_v, [gidx[c]], gw[c] * invd)

                def tok(t, c0):
                    r = t * C
                    a = [plsc.load_gather(
                        w_v, [jnp.full((LANES,), r + c, jnp.int32)])
                        for c in range(C)]

                    def hblk(h, c1):
                        sl = pl.ds(h * LANES, LANES)
                        v = (rows_v[b, r, sl] * a[0] + rows_v[b, r + 1, sl] * a[1]
                             + rows_v[b, r + 2, sl] * a[2]
                             + rows_v[b, r + 3, sl] * a[3])
                        out_v[b, t, sl] = v
                        return c1

                    return lax.fori_loop(0, HB, hblk, c0)

                lax.fori_loop(0, CHUNK, tok, 0)

            pltpu.async_copy(out_v.at[b], out_hbm.at[pl.ds(tok0, CHUNK)],
                             semos[b])

        start(0, 0)

        def body(i, carry):
            s0 = 2 * i
            start(s0 + 1, 1)
            wait(0)
            compute(s0, 0)

            @pl.when(s0 + 2 < NSTEP)
            def _():
                start(s0 + 2, 0)

            wait(1)
            compute(s0 + 1, 1)
            return carry

        lax.fori_loop(0, NSTEP // 2, body, 0)

        for b in (0, 1):
            pltpu.make_async_copy(
                out_v.at[b],
                out_hbm.at[pl.ds(base + (NSTEP - 2 + b) * CHUNK, CHUNK)],
                semos[b]).wait()

    return k(ids_flat, word_table)


def _tc_ln(pooled, ttf3, pos_table, type_table, ln_gamma, ln_beta):
    def body(pooled_ref, ttf_ref, pos_ref, ty_ref, gam_ref, bet_ref, out_ref):
        x = pooled_ref[...]
        ptt = ttf_ref[0, 0]
        dty = ty_ref[1] - ty_ref[0]
        ty = ty_ref[0][None, :] + ptt[:, None] * dty[None, :]
        e = x + pos_ref[...] + ty
        u = jnp.mean(e, axis=-1, keepdims=True)
        s = jnp.mean((e - u) ** 2, axis=-1, keepdims=True)
        xn = (e - u) * lax.rsqrt(s + EPS)
        out_ref[...] = gam_ref[...][None, :] * xn + bet_ref[...][None, :]

    grid = (NSB, B)
    return pl.pallas_call(
        body,
        grid=grid,
        in_specs=[
            pl.BlockSpec((SBLK, HID), lambda si, bi: (bi * NSB + si, 0)),
            pl.BlockSpec((1, 1, SBLK), lambda si, bi: (bi * NSB + si, 0, 0)),
            pl.BlockSpec((SBLK, HID), lambda si, bi: (si, 0)),
            pl.BlockSpec((TYPES, HID), lambda si, bi: (0, 0)),
            pl.BlockSpec((HID,), lambda si, bi: (0,)),
            pl.BlockSpec((HID,), lambda si, bi: (0,)),
        ],
        out_specs=pl.BlockSpec((SBLK, HID), lambda si, bi: (bi * NSB + si, 0)),
        out_shape=jax.ShapeDtypeStruct((N, HID), jnp.float32),
    )(pooled, ttf3, pos_table, type_table, ln_gamma, ln_beta)


def kernel(input_ids, token_type_ids, token_ids, word_table, pos_table,
           type_table, ln_gamma, ln_beta):
    ids_flat = token_ids.astype(jnp.int32).reshape(N * C)
    pooled = _sc_pool(ids_flat, word_table)
    ttf3 = token_type_ids.astype(jnp.float32).reshape(B * NSB, 1, SBLK)
    out = _tc_ln(pooled, ttf3, pos_table, type_table, ln_gamma, ln_beta)
    return out.reshape(B, S, HID)
